# baseline (speedup 1.0000x reference)
import jax
import jax.numpy as jnp
from jax import lax
from jax.experimental import pallas as pl
from jax.experimental.pallas import tpu as pltpu
from jax.experimental.pallas import tpu_sc as plsc

_N = 10000
_E = 320000
_D = 128
_H = 128
_NORM = 100.0

_PARTS = ((0, 128000), (128000, 128000), (256000, 64000))
_NC = 2
_NS = 16
_NW = _NC * _NS
_CH = 200
_CHS = 200
_NP = 10240
_RPT = _NP // _NS
_RST = 128

_f32 = jnp.float32


_bf16 = jnp.bfloat16


def _pre_body(h_ref, wa_ref, wb_ref, b1_ref, ei_ref, a_ref, b_ref,
              row_ref, col_ref):
    hb = h_ref[...]
    a_ref[...] = (jnp.dot(hb, wa_ref[...], preferred_element_type=_f32)
                  + b1_ref[...])
    b_ref[...] = jnp.dot(hb, wb_ref[...], preferred_element_type=_f32)
    row_ref[...] = ei_ref[0, :]
    col_ref[...] = ei_ref[1, :]


def _precompute(h, wa, wb, be1_2d, edge_index):
    bn = 1000
    beb = 32768
    epad = beb * (_N // bn)
    return pl.pallas_call(
        _pre_body,
        grid=(_N // bn,),
        in_specs=[
            pl.BlockSpec((bn, _D), lambda i: (i, 0)),
            pl.BlockSpec((_D, _H), lambda i: (0, 0)),
            pl.BlockSpec((_D, _H), lambda i: (0, 0)),
            pl.BlockSpec((1, _H), lambda i: (0, 0)),
            pl.BlockSpec((2, beb), lambda i: (0, i)),
        ],
        out_specs=[
            pl.BlockSpec((bn, _H), lambda i: (i, 0)),
            pl.BlockSpec((bn, _H), lambda i: (i, 0)),
            pl.BlockSpec((beb,), lambda i: (i,)),
            pl.BlockSpec((beb,), lambda i: (i,)),
        ],
        out_shape=[
            jax.ShapeDtypeStruct((_N, _H), _f32),
            jax.ShapeDtypeStruct((_N, _H), _f32),
            jax.ShapeDtypeStruct((epad,), jnp.int32),
            jax.ShapeDtypeStruct((epad,), jnp.int32),
        ],
    )(h, wa, wb, be1_2d, edge_index)


def _make_gather_body(off, epw):
    nch = epw // _CH
    assert nch % 2 == 0 and nch >= 4

    def _gather_body(a_hbm, b_hbm, row_hbm, col_hbm, g1_hbm, g2_hbm,
                     i1a, i2a, ba1, ba2, i1b, i2b, bb1, bb2,
                     gsem1, gsem2, wsem_a, wsem_b):
        wid = lax.axis_index("s") * _NC + lax.axis_index("c")
        base_w = wid * epw

        def chunk(k, idx1_v, idx2_v, buf1_v, buf2_v, wsem, drain):
            base = base_w + k * _CH
            if drain:
                pltpu.make_async_copy(buf1_v, g1_hbm.at[pl.ds(base, _CH)],
                                      wsem).wait()
                pltpu.make_async_copy(buf2_v, g2_hbm.at[pl.ds(base, _CH)],
                                      wsem).wait()
            pltpu.sync_copy(row_hbm.at[pl.ds(off + base, _CH)], idx1_v)
            pltpu.sync_copy(col_hbm.at[pl.ds(off + base, _CH)], idx2_v)
            cp1 = pltpu.async_copy(a_hbm.at[idx1_v], buf1_v, gsem1)
            cp2 = pltpu.async_copy(b_hbm.at[idx2_v], buf2_v, gsem2)
            cp1.wait()
            cp2.wait()
            pltpu.async_copy(buf1_v, g1_hbm.at[pl.ds(base, _CH)], wsem)
            pltpu.async_copy(buf2_v, g2_hbm.at[pl.ds(base, _CH)], wsem)

        chunk(0, i1a, i2a, ba1, ba2, wsem_a, False)
        chunk(1, i1b, i2b, bb1, bb2, wsem_b, False)

        def pair(j, carry):
            chunk(2 * j, i1a, i2a, ba1, ba2, wsem_a, True)
            chunk(2 * j + 1, i1b, i2b, bb1, bb2, wsem_b, True)
            return carry

        lax.fori_loop(1, nch // 2, pair, 0)
        pltpu.make_async_copy(ba1, g1_hbm.at[pl.ds(base_w, _CH)],
                              wsem_a).wait()
        pltpu.make_async_copy(ba2, g2_hbm.at[pl.ds(base_w, _CH)],
                              wsem_a).wait()
        pltpu.make_async_copy(bb1, g1_hbm.at[pl.ds(base_w, _CH)],
                              wsem_b).wait()
        pltpu.make_async_copy(bb2, g2_hbm.at[pl.ds(base_w, _CH)],
                              wsem_b).wait()

    return _gather_body


def _sc_gather(a, b, row, col, off, size):
    mesh = plsc.VectorSubcoreMesh(core_axis_name="c", subcore_axis_name="s")
    fn = pl.kernel(
        _make_gather_body(off, size // _NW),
        out_type=[jax.ShapeDtypeStruct((size, _H), _f32)] * 2,
        mesh=mesh,
        scratch_types=[
            pltpu.VMEM((_CH,), jnp.int32),
            pltpu.VMEM((_CH,), jnp.int32),
            pltpu.VMEM((_CH, _H), _f32),
            pltpu.VMEM((_CH, _H), _f32),
            pltpu.VMEM((_CH,), jnp.int32),
            pltpu.VMEM((_CH,), jnp.int32),
            pltpu.VMEM((_CH, _H), _f32),
            pltpu.VMEM((_CH, _H), _f32),
            pltpu.SemaphoreType.DMA,
            pltpu.SemaphoreType.DMA,
            pltpu.SemaphoreType.DMA,
            pltpu.SemaphoreType.DMA,
        ],
    )
    return fn(a, b, row, col)


def _silu(x):
    return x * (0.5 + 0.5 * jnp.tanh(0.5 * x))


def _edge_body(g1_ref, g2_ref, w2_ref, b2_ref, out_ref):
    x = _silu(g1_ref[...] + g2_ref[...])
    y = (jnp.dot(x.astype(_bf16), w2_ref[...].astype(_bf16),
                 preferred_element_type=_f32) + b2_ref[...])
    out_ref[...] = _silu(y)


def _edge_mlp(g1, g2, We2, be2_2d):
    be = 2000
    return pl.pallas_call(
        _edge_body,
        grid=(g1.shape[0] // be,),
        in_specs=[
            pl.BlockSpec((be, _H), lambda i: (i, 0)),
            pl.BlockSpec((be, _H), lambda i: (i, 0)),
            pl.BlockSpec((_H, _H), lambda i: (0, 0)),
            pl.BlockSpec((1, _H), lambda i: (0, 0)),
        ],
        out_specs=pl.BlockSpec((be, _H), lambda i: (i, 0)),
        out_shape=jax.ShapeDtypeStruct((g1.shape[0], _H), _f32),
    )(g1, g2, We2, be2_2d)


def _make_scatter_body(off, epw):
    nchs = epw // _CHS

    def _scatter_body(e2_hbm, row_hbm, out_hbm, idx_a, ebuf_a, zbuf_v,
                      acc_sh):
        c = lax.axis_index("c")
        s = lax.axis_index("s")
        wid = s * _NC + c

        def zfill(t, carry):
            zbuf_v[t // 8, pl.ds((t % 8) * 16, 16)] = jnp.zeros((16,), _f32)
            return carry

        lax.fori_loop(0, _RST * 8, zfill, 0)

        def zslice(j, carry):
            pltpu.sync_copy(zbuf_v,
                            acc_sh.at[pl.ds(s * _RPT + j * _RST, _RST)])
            return carry

        lax.fori_loop(0, _RPT // _RST, zslice, 0)
        plsc.subcore_barrier()

        base_w = wid * epw

        def step(k, carry):
            base = base_w + k * _CHS
            pltpu.sync_copy(row_hbm.at[pl.ds(off + base, _CHS)], idx_a)
            pltpu.sync_copy(e2_hbm.at[pl.ds(base, _CHS)], ebuf_a)
            pltpu.sync_copy(ebuf_a, acc_sh.at[idx_a], add=True)
            return carry

        lax.fori_loop(0, nchs, step, 0)
        plsc.subcore_barrier()

        def wout(j, carry):
            r0 = s * _RPT + j * _RST
            pltpu.sync_copy(acc_sh.at[pl.ds(r0, _RST)], zbuf_v)
            pltpu.sync_copy(zbuf_v, out_hbm.at[c, pl.ds(r0, _RST)])
            return carry

        lax.fori_loop(0, _RPT // _RST, wout, 0)

    return _scatter_body


def _sc_scatter(e2, row, off):
    mesh = plsc.VectorSubcoreMesh(core_axis_name="c", subcore_axis_name="s")
    fn = pl.kernel(
        _make_scatter_body(off, e2.shape[0] // _NW),
        out_type=jax.ShapeDtypeStruct((_NC, _NP, _H), _f32),
        mesh=mesh,
        scratch_types=[
            pltpu.VMEM((_CHS,), jnp.int32),
            pltpu.VMEM((_CHS, _H), _f32),
            pltpu.VMEM((_RST, _H), _f32),
            pltpu.VMEM_SHARED((_NP, _H), _f32),
        ],
    )
    return fn(e2, row)


def _node_body(h_ref, p0_ref, p1_ref, p2_ref, wa_ref, wb_ref, b1_ref, w2_ref,
               b2_ref, out_ref):
    hb = h_ref[...]
    agg = (p0_ref[0] + p0_ref[1] + p1_ref[0] + p1_ref[1]
           + p2_ref[0] + p2_ref[1]) * (1.0 / _NORM)
    x = (jnp.dot(hb, wa_ref[...], preferred_element_type=_f32)
         + jnp.dot(agg, wb_ref[...], preferred_element_type=_f32)
         + b1_ref[...])
    x = _silu(x)
    z = jnp.dot(x, w2_ref[...], preferred_element_type=_f32) + b2_ref[...]
    out_ref[...] = hb + z


def _node_mlp(h, p0, p1, p2, wa, wb, bn1_2d, Wn2, bn2_2d):
    bn = 1000
    pspec = pl.BlockSpec((_NC, bn, _H), lambda i: (0, i, 0))
    return pl.pallas_call(
        _node_body,
        grid=(_N // bn,),
        in_specs=[
            pl.BlockSpec((bn, _D), lambda i: (i, 0)),
            pspec,
            pspec,
            pspec,
            pl.BlockSpec((_D, _H), lambda i: (0, 0)),
            pl.BlockSpec((_H, _H), lambda i: (0, 0)),
            pl.BlockSpec((1, _H), lambda i: (0, 0)),
            pl.BlockSpec((_H, _D), lambda i: (0, 0)),
            pl.BlockSpec((1, _D), lambda i: (0, 0)),
        ],
        out_specs=pl.BlockSpec((bn, _D), lambda i: (i, 0)),
        out_shape=jax.ShapeDtypeStruct((_N, _D), _f32),
    )(h, p0, p1, p2, wa, wb, bn1_2d, Wn2, bn2_2d)


def kernel(h, edge_index, We1, be1, We2, be2, Wn1, bn1, Wn2, bn2):
    a, b, row, col = _precompute(h, We1[:_D], We1[_D:], be1.reshape(1, _H),
                                 edge_index)
    be2_2d = be2.reshape(1, _H)
    partials = []
    for off, size in _PARTS:
        g1, g2 = _sc_gather(a, b, row, col, off, size)
        e2 = _edge_mlp(g1, g2, We2, be2_2d)
        partials.append(_sc_scatter(e2, row, off))
    return _node_mlp(h, partials[0], partials[1], partials[2], Wn1[:_D],
                     Wn1[_D:], bn1.reshape(1, _H), Wn2, bn2.reshape(1, _D))


# speedup vs baseline: 1.0607x; 1.0607x over previous
import jax
import jax.numpy as jnp
from jax import lax
from jax.experimental import pallas as pl
from jax.experimental.pallas import tpu as pltpu
from jax.experimental.pallas import tpu_sc as plsc

_N = 10000
_E = 320000
_D = 128
_H = 128
_NORM = 100.0

_PARTS = ((0, 160000), (160000, 160000))
_NC = 2
_NS = 16
_NW = _NC * _NS
_CH = 200
_CHS = 200
_NP = 10240
_RPT = _NP // _NS
_RST = 128

_f32 = jnp.float32


_bf16 = jnp.bfloat16


def _pre_body(h_ref, wa_ref, wb_ref, b1_ref, ei_ref, a_ref, b_ref,
              row_ref, col_ref):
    hb = h_ref[...]
    a_ref[...] = (jnp.dot(hb, wa_ref[...], preferred_element_type=_f32)
                  + b1_ref[...])
    b_ref[...] = jnp.dot(hb, wb_ref[...], preferred_element_type=_f32)
    row_ref[...] = ei_ref[0, :]
    col_ref[...] = ei_ref[1, :]


def _precompute(h, wa, wb, be1_2d, edge_index):
    bn = 1000
    beb = 32768
    epad = beb * (_N // bn)
    return pl.pallas_call(
        _pre_body,
        grid=(_N // bn,),
        in_specs=[
            pl.BlockSpec((bn, _D), lambda i: (i, 0)),
            pl.BlockSpec((_D, _H), lambda i: (0, 0)),
            pl.BlockSpec((_D, _H), lambda i: (0, 0)),
            pl.BlockSpec((1, _H), lambda i: (0, 0)),
            pl.BlockSpec((2, beb), lambda i: (0, i)),
        ],
        out_specs=[
            pl.BlockSpec((bn, _H), lambda i: (i, 0)),
            pl.BlockSpec((bn, _H), lambda i: (i, 0)),
            pl.BlockSpec((beb,), lambda i: (i,)),
            pl.BlockSpec((beb,), lambda i: (i,)),
        ],
        out_shape=[
            jax.ShapeDtypeStruct((_N, _H), _f32),
            jax.ShapeDtypeStruct((_N, _H), _f32),
            jax.ShapeDtypeStruct((epad,), jnp.int32),
            jax.ShapeDtypeStruct((epad,), jnp.int32),
        ],
    )(h, wa, wb, be1_2d, edge_index)


def _make_gather_body(off, epw):
    nch = epw // _CH
    assert nch >= 4

    def _gather_body(a_hbm, b_hbm, row_hbm, col_hbm, g1_hbm, g2_hbm,
                     i1a, i2a, ba1, ba2, i1b, i2b, bb1, bb2,
                     gsem1, gsem2, wsem_a, wsem_b):
        wid = lax.axis_index("s") * _NC + lax.axis_index("c")
        base_w = wid * epw

        def chunk(k, idx1_v, idx2_v, buf1_v, buf2_v, wsem, drain):
            base = base_w + k * _CH
            if drain:
                pltpu.make_async_copy(buf1_v, g1_hbm.at[pl.ds(base, _CH)],
                                      wsem).wait()
                pltpu.make_async_copy(buf2_v, g2_hbm.at[pl.ds(base, _CH)],
                                      wsem).wait()
            pltpu.sync_copy(row_hbm.at[pl.ds(off + base, _CH)], idx1_v)
            pltpu.sync_copy(col_hbm.at[pl.ds(off + base, _CH)], idx2_v)
            cp1 = pltpu.async_copy(a_hbm.at[idx1_v], buf1_v, gsem1)
            cp2 = pltpu.async_copy(b_hbm.at[idx2_v], buf2_v, gsem2)
            cp1.wait()
            cp2.wait()
            pltpu.async_copy(buf1_v, g1_hbm.at[pl.ds(base, _CH)], wsem)
            pltpu.async_copy(buf2_v, g2_hbm.at[pl.ds(base, _CH)], wsem)

        chunk(0, i1a, i2a, ba1, ba2, wsem_a, False)
        chunk(1, i1b, i2b, bb1, bb2, wsem_b, False)

        def pair(j, carry):
            chunk(2 * j, i1a, i2a, ba1, ba2, wsem_a, True)
            chunk(2 * j + 1, i1b, i2b, bb1, bb2, wsem_b, True)
            return carry

        lax.fori_loop(1, nch // 2, pair, 0)
        if nch % 2:
            chunk(nch - 1, i1a, i2a, ba1, ba2, wsem_a, True)
        pltpu.make_async_copy(ba1, g1_hbm.at[pl.ds(base_w, _CH)],
                              wsem_a).wait()
        pltpu.make_async_copy(ba2, g2_hbm.at[pl.ds(base_w, _CH)],
                              wsem_a).wait()
        pltpu.make_async_copy(bb1, g1_hbm.at[pl.ds(base_w, _CH)],
                              wsem_b).wait()
        pltpu.make_async_copy(bb2, g2_hbm.at[pl.ds(base_w, _CH)],
                              wsem_b).wait()

    return _gather_body


def _sc_gather(a, b, row, col, off, size):
    mesh = plsc.VectorSubcoreMesh(core_axis_name="c", subcore_axis_name="s")
    fn = pl.kernel(
        _make_gather_body(off, size // _NW),
        out_type=[jax.ShapeDtypeStruct((size, _H), _f32)] * 2,
        mesh=mesh,
        scratch_types=[
            pltpu.VMEM((_CH,), jnp.int32),
            pltpu.VMEM((_CH,), jnp.int32),
            pltpu.VMEM((_CH, _H), _f32),
            pltpu.VMEM((_CH, _H), _f32),
            pltpu.VMEM((_CH,), jnp.int32),
            pltpu.VMEM((_CH,), jnp.int32),
            pltpu.VMEM((_CH, _H), _f32),
            pltpu.VMEM((_CH, _H), _f32),
            pltpu.SemaphoreType.DMA,
            pltpu.SemaphoreType.DMA,
            pltpu.SemaphoreType.DMA,
            pltpu.SemaphoreType.DMA,
        ],
    )
    return fn(a, b, row, col)


def _silu(x):
    return x * (0.5 + 0.5 * jnp.tanh(0.5 * x))


def _edge_body(g1_ref, g2_ref, w2_ref, b2_ref, out_ref):
    x = _silu(g1_ref[...] + g2_ref[...])
    y = (jnp.dot(x.astype(_bf16), w2_ref[...].astype(_bf16),
                 preferred_element_type=_f32) + b2_ref[...])
    out_ref[...] = _silu(y)


def _edge_mlp(g1, g2, We2, be2_2d):
    be = 2000
    return pl.pallas_call(
        _edge_body,
        grid=(g1.shape[0] // be,),
        in_specs=[
            pl.BlockSpec((be, _H), lambda i: (i, 0)),
            pl.BlockSpec((be, _H), lambda i: (i, 0)),
            pl.BlockSpec((_H, _H), lambda i: (0, 0)),
            pl.BlockSpec((1, _H), lambda i: (0, 0)),
        ],
        out_specs=pl.BlockSpec((be, _H), lambda i: (i, 0)),
        out_shape=jax.ShapeDtypeStruct((g1.shape[0], _H), _f32),
    )(g1, g2, We2, be2_2d)


def _make_scatter_body(off, epw):
    nchs = epw // _CHS

    def _scatter_body(e2_hbm, row_hbm, out_hbm, idx_a, ebuf_a, zbuf_v,
                      acc_sh):
        c = lax.axis_index("c")
        s = lax.axis_index("s")
        wid = s * _NC + c

        def zfill(t, carry):
            zbuf_v[t // 8, pl.ds((t % 8) * 16, 16)] = jnp.zeros((16,), _f32)
            return carry

        lax.fori_loop(0, _RST * 8, zfill, 0)

        def zslice(j, carry):
            pltpu.sync_copy(zbuf_v,
                            acc_sh.at[pl.ds(s * _RPT + j * _RST, _RST)])
            return carry

        lax.fori_loop(0, _RPT // _RST, zslice, 0)
        plsc.subcore_barrier()

        base_w = wid * epw

        def step(k, carry):
            base = base_w + k * _CHS
            pltpu.sync_copy(row_hbm.at[pl.ds(off + base, _CHS)], idx_a)
            pltpu.sync_copy(e2_hbm.at[pl.ds(base, _CHS)], ebuf_a)
            pltpu.sync_copy(ebuf_a, acc_sh.at[idx_a], add=True)
            return carry

        lax.fori_loop(0, nchs, step, 0)
        plsc.subcore_barrier()

        def wout(j, carry):
            r0 = s * _RPT + j * _RST
            pltpu.sync_copy(acc_sh.at[pl.ds(r0, _RST)], zbuf_v)
            pltpu.sync_copy(zbuf_v, out_hbm.at[c, pl.ds(r0, _RST)])
            return carry

        lax.fori_loop(0, _RPT // _RST, wout, 0)

    return _scatter_body


def _sc_scatter(e2, row, off):
    mesh = plsc.VectorSubcoreMesh(core_axis_name="c", subcore_axis_name="s")
    fn = pl.kernel(
        _make_scatter_body(off, e2.shape[0] // _NW),
        out_type=jax.ShapeDtypeStruct((_NC, _NP, _H), _f32),
        mesh=mesh,
        scratch_types=[
            pltpu.VMEM((_CHS,), jnp.int32),
            pltpu.VMEM((_CHS, _H), _f32),
            pltpu.VMEM((_RST, _H), _f32),
            pltpu.VMEM_SHARED((_NP, _H), _f32),
        ],
    )
    return fn(e2, row)


def _node_body(h_ref, p0_ref, p1_ref, wa_ref, wb_ref, b1_ref, w2_ref,
               b2_ref, out_ref):
    hb = h_ref[...]
    agg = (p0_ref[0] + p0_ref[1] + p1_ref[0] + p1_ref[1]) * (1.0 / _NORM)
    x = (jnp.dot(hb, wa_ref[...], preferred_element_type=_f32)
         + jnp.dot(agg, wb_ref[...], preferred_element_type=_f32)
         + b1_ref[...])
    x = _silu(x)
    z = jnp.dot(x, w2_ref[...], preferred_element_type=_f32) + b2_ref[...]
    out_ref[...] = hb + z


def _node_mlp(h, p0, p1, wa, wb, bn1_2d, Wn2, bn2_2d):
    bn = 1000
    pspec = pl.BlockSpec((_NC, bn, _H), lambda i: (0, i, 0))
    return pl.pallas_call(
        _node_body,
        grid=(_N // bn,),
        in_specs=[
            pl.BlockSpec((bn, _D), lambda i: (i, 0)),
            pspec,
            pspec,
            pl.BlockSpec((_D, _H), lambda i: (0, 0)),
            pl.BlockSpec((_H, _H), lambda i: (0, 0)),
            pl.BlockSpec((1, _H), lambda i: (0, 0)),
            pl.BlockSpec((_H, _D), lambda i: (0, 0)),
            pl.BlockSpec((1, _D), lambda i: (0, 0)),
        ],
        out_specs=pl.BlockSpec((bn, _D), lambda i: (i, 0)),
        out_shape=jax.ShapeDtypeStruct((_N, _D), _f32),
    )(h, p0, p1, wa, wb, bn1_2d, Wn2, bn2_2d)


def kernel(h, edge_index, We1, be1, We2, be2, Wn1, bn1, Wn2, bn2):
    a, b, row, col = _precompute(h, We1[:_D], We1[_D:], be1.reshape(1, _H),
                                 edge_index)
    be2_2d = be2.reshape(1, _H)
    partials = []
    for off, size in _PARTS:
        g1, g2 = _sc_gather(a, b, row, col, off, size)
        e2 = _edge_mlp(g1, g2, We2, be2_2d)
        partials.append(_sc_scatter(e2, row, off))
    return _node_mlp(h, partials[0], partials[1], Wn1[:_D],
                     Wn1[_D:], bn1.reshape(1, _H), Wn2, bn2.reshape(1, _D))


# speedup vs baseline: 1.0625x; 1.0016x over previous
import jax
import jax.numpy as jnp
from jax import lax
from jax.experimental import pallas as pl
from jax.experimental.pallas import tpu as pltpu
from jax.experimental.pallas import tpu_sc as plsc

_N = 10000
_E = 320000
_D = 128
_H = 128
_NORM = 100.0

_PARTS = ((0, 160000), (160000, 160000))
_NC = 2
_NS = 16
_NW = _NC * _NS
_CH = 200
_CHS = 200
_NP = 10240
_RPT = _NP // _NS
_RST = 128

_f32 = jnp.float32


_bf16 = jnp.bfloat16


def _pre_body(h_ref, wa_ref, wb_ref, b1_ref, ei_ref, a_ref, b_ref,
              row_ref, col_ref):
    hb = h_ref[...]
    a_ref[...] = (jnp.dot(hb, wa_ref[...], preferred_element_type=_f32)
                  + b1_ref[...])
    b_ref[...] = jnp.dot(hb, wb_ref[...], preferred_element_type=_f32)
    row_ref[...] = ei_ref[0, :]
    col_ref[...] = ei_ref[1, :]


def _precompute(h, wa, wb, be1_2d, edge_index):
    bn = 1000
    beb = 32768
    epad = beb * (_N // bn)
    return pl.pallas_call(
        _pre_body,
        grid=(_N // bn,),
        in_specs=[
            pl.BlockSpec((bn, _D), lambda i: (i, 0)),
            pl.BlockSpec((_D, _H), lambda i: (0, 0)),
            pl.BlockSpec((_D, _H), lambda i: (0, 0)),
            pl.BlockSpec((1, _H), lambda i: (0, 0)),
            pl.BlockSpec((2, beb), lambda i: (0, i)),
        ],
        out_specs=[
            pl.BlockSpec((bn, _H), lambda i: (i, 0)),
            pl.BlockSpec((bn, _H), lambda i: (i, 0)),
            pl.BlockSpec((beb,), lambda i: (i,)),
            pl.BlockSpec((beb,), lambda i: (i,)),
        ],
        out_shape=[
            jax.ShapeDtypeStruct((_N, _H), _f32),
            jax.ShapeDtypeStruct((_N, _H), _f32),
            jax.ShapeDtypeStruct((epad,), jnp.int32),
            jax.ShapeDtypeStruct((epad,), jnp.int32),
        ],
    )(h, wa, wb, be1_2d, edge_index)


def _make_gather_body(off, epw):
    nch = epw // _CH
    assert nch >= 4

    def _gather_body(a_hbm, b_hbm, row_hbm, col_hbm, g1_hbm, g2_hbm,
                     i1a, i2a, ba1, ba2, i1b, i2b, bb1, bb2,
                     gsem1, gsem2, wsem_a, wsem_b):
        wid = lax.axis_index("s") * _NC + lax.axis_index("c")
        base_w = wid * epw

        def chunk(k, idx1_v, idx2_v, buf1_v, buf2_v, wsem, drain):
            base = base_w + k * _CH
            if drain:
                pltpu.make_async_copy(buf1_v, g1_hbm.at[pl.ds(base, _CH)],
                                      wsem).wait()
                pltpu.make_async_copy(buf2_v, g2_hbm.at[pl.ds(base, _CH)],
                                      wsem).wait()
            pltpu.sync_copy(row_hbm.at[pl.ds(off + base, _CH)], idx1_v)
            pltpu.sync_copy(col_hbm.at[pl.ds(off + base, _CH)], idx2_v)
            cp1 = pltpu.async_copy(a_hbm.at[idx1_v], buf1_v, gsem1)
            cp2 = pltpu.async_copy(b_hbm.at[idx2_v], buf2_v, gsem2)
            cp1.wait()
            cp2.wait()
            pltpu.async_copy(buf1_v, g1_hbm.at[pl.ds(base, _CH)], wsem)
            pltpu.async_copy(buf2_v, g2_hbm.at[pl.ds(base, _CH)], wsem)

        chunk(0, i1a, i2a, ba1, ba2, wsem_a, False)
        chunk(1, i1b, i2b, bb1, bb2, wsem_b, False)

        def pair(j, carry):
            chunk(2 * j, i1a, i2a, ba1, ba2, wsem_a, True)
            chunk(2 * j + 1, i1b, i2b, bb1, bb2, wsem_b, True)
            return carry

        lax.fori_loop(1, nch // 2, pair, 0)
        if nch % 2:
            chunk(nch - 1, i1a, i2a, ba1, ba2, wsem_a, True)
        pltpu.make_async_copy(ba1, g1_hbm.at[pl.ds(base_w, _CH)],
                              wsem_a).wait()
        pltpu.make_async_copy(ba2, g2_hbm.at[pl.ds(base_w, _CH)],
                              wsem_a).wait()
        pltpu.make_async_copy(bb1, g1_hbm.at[pl.ds(base_w, _CH)],
                              wsem_b).wait()
        pltpu.make_async_copy(bb2, g2_hbm.at[pl.ds(base_w, _CH)],
                              wsem_b).wait()

    return _gather_body


def _sc_gather(a, b, row, col, off, size):
    mesh = plsc.VectorSubcoreMesh(core_axis_name="c", subcore_axis_name="s")
    fn = pl.kernel(
        _make_gather_body(off, size // _NW),
        out_type=[jax.ShapeDtypeStruct((size, _H), _f32)] * 2,
        mesh=mesh,
        scratch_types=[
            pltpu.VMEM((_CH,), jnp.int32),
            pltpu.VMEM((_CH,), jnp.int32),
            pltpu.VMEM((_CH, _H), _f32),
            pltpu.VMEM((_CH, _H), _f32),
            pltpu.VMEM((_CH,), jnp.int32),
            pltpu.VMEM((_CH,), jnp.int32),
            pltpu.VMEM((_CH, _H), _f32),
            pltpu.VMEM((_CH, _H), _f32),
            pltpu.SemaphoreType.DMA,
            pltpu.SemaphoreType.DMA,
            pltpu.SemaphoreType.DMA,
            pltpu.SemaphoreType.DMA,
        ],
    )
    return fn(a, b, row, col)


def _silu(x):
    return x * (0.5 + 0.5 * jnp.tanh(0.5 * x))


def _edge_body(g1_ref, g2_ref, w2_ref, b2_ref, out_ref):
    x = _silu(g1_ref[...] + g2_ref[...])
    y = (jnp.dot(x.astype(_bf16), w2_ref[...].astype(_bf16),
                 preferred_element_type=_f32) + b2_ref[...])
    out_ref[...] = _silu(y)


def _edge_mlp(g1, g2, We2, be2_2d):
    be = 2000
    return pl.pallas_call(
        _edge_body,
        grid=(g1.shape[0] // be,),
        in_specs=[
            pl.BlockSpec((be, _H), lambda i: (i, 0)),
            pl.BlockSpec((be, _H), lambda i: (i, 0)),
            pl.BlockSpec((_H, _H), lambda i: (0, 0)),
            pl.BlockSpec((1, _H), lambda i: (0, 0)),
        ],
        out_specs=pl.BlockSpec((be, _H), lambda i: (i, 0)),
        out_shape=jax.ShapeDtypeStruct((g1.shape[0], _H), _f32),
    )(g1, g2, We2, be2_2d)


def _make_scatter_body(off, epw):
    nchs = epw // _CHS

    def _scatter_body(e2_hbm, row_hbm, out_hbm, idx_a, ebuf_a, zbuf_v,
                      acc_sh):
        c = lax.axis_index("c")
        s = lax.axis_index("s")
        wid = s * _NC + c

        def zfill(t, carry):
            zbuf_v[t // 8, pl.ds((t % 8) * 16, 16)] = jnp.zeros((16,), _f32)
            return carry

        lax.fori_loop(0, _RST * 8, zfill, 0)

        def zslice(j, carry):
            pltpu.sync_copy(zbuf_v,
                            acc_sh.at[pl.ds(s * _RPT + j * _RST, _RST)])
            return carry

        lax.fori_loop(0, _RPT // _RST, zslice, 0)
        plsc.subcore_barrier()

        base_w = wid * epw

        def step(k, carry):
            base = base_w + k * _CHS
            pltpu.sync_copy(row_hbm.at[pl.ds(off + base, _CHS)], idx_a)
            pltpu.sync_copy(e2_hbm.at[pl.ds(base, _CHS)], ebuf_a)
            pltpu.sync_copy(ebuf_a, acc_sh.at[idx_a], add=True)
            return carry

        lax.fori_loop(0, nchs, step, 0)
        plsc.subcore_barrier()

        r0 = s * _RPT
        pltpu.sync_copy(acc_sh.at[pl.ds(r0, _RPT)],
                        out_hbm.at[c, pl.ds(r0, _RPT)])

    return _scatter_body


def _sc_scatter(e2, row, off):
    mesh = plsc.VectorSubcoreMesh(core_axis_name="c", subcore_axis_name="s")
    fn = pl.kernel(
        _make_scatter_body(off, e2.shape[0] // _NW),
        out_type=jax.ShapeDtypeStruct((_NC, _NP, _H), _f32),
        mesh=mesh,
        scratch_types=[
            pltpu.VMEM((_CHS,), jnp.int32),
            pltpu.VMEM((_CHS, _H), _f32),
            pltpu.VMEM((_RST, _H), _f32),
            pltpu.VMEM_SHARED((_NP, _H), _f32),
        ],
    )
    return fn(e2, row)


def _node_body(h_ref, p0_ref, p1_ref, wa_ref, wb_ref, b1_ref, w2_ref,
               b2_ref, out_ref):
    hb = h_ref[...]
    agg = (p0_ref[0] + p0_ref[1] + p1_ref[0] + p1_ref[1]) * (1.0 / _NORM)
    x = (jnp.dot(hb, wa_ref[...], preferred_element_type=_f32)
         + jnp.dot(agg, wb_ref[...], preferred_element_type=_f32)
         + b1_ref[...])
    x = _silu(x)
    z = jnp.dot(x, w2_ref[...], preferred_element_type=_f32) + b2_ref[...]
    out_ref[...] = hb + z


def _node_mlp(h, p0, p1, wa, wb, bn1_2d, Wn2, bn2_2d):
    bn = 1000
    pspec = pl.BlockSpec((_NC, bn, _H), lambda i: (0, i, 0))
    return pl.pallas_call(
        _node_body,
        grid=(_N // bn,),
        in_specs=[
            pl.BlockSpec((bn, _D), lambda i: (i, 0)),
            pspec,
            pspec,
            pl.BlockSpec((_D, _H), lambda i: (0, 0)),
            pl.BlockSpec((_H, _H), lambda i: (0, 0)),
            pl.BlockSpec((1, _H), lambda i: (0, 0)),
            pl.BlockSpec((_H, _D), lambda i: (0, 0)),
            pl.BlockSpec((1, _D), lambda i: (0, 0)),
        ],
        out_specs=pl.BlockSpec((bn, _D), lambda i: (i, 0)),
        out_shape=jax.ShapeDtypeStruct((_N, _D), _f32),
    )(h, p0, p1, wa, wb, bn1_2d, Wn2, bn2_2d)


def kernel(h, edge_index, We1, be1, We2, be2, Wn1, bn1, Wn2, bn2):
    a, b, row, col = _precompute(h, We1[:_D], We1[_D:], be1.reshape(1, _H),
                                 edge_index)
    be2_2d = be2.reshape(1, _H)
    partials = []
    for off, size in _PARTS:
        g1, g2 = _sc_gather(a, b, row, col, off, size)
        e2 = _edge_mlp(g1, g2, We2, be2_2d)
        partials.append(_sc_scatter(e2, row, off))
    return _node_mlp(h, partials[0], partials[1], Wn1[:_D],
                     Wn1[_D:], bn1.reshape(1, _H), Wn2, bn2.reshape(1, _D))
